# Initial kernel scaffold; baseline (speedup 1.0000x reference)
import jax
import jax.numpy as jnp
from jax.experimental import pallas as pl


def kernel(coordinates, planes_features):
    raise NotImplementedError("write your pallas kernel here")


# speedup vs baseline: 60.6097x; 60.6097x over previous
import functools

import jax
import jax.numpy as jnp
from jax import lax
from jax.experimental import pallas as pl
from jax.experimental.pallas import tpu as pltpu
from jax.experimental.pallas import tpu_sc as plsc

N_PTS = 524288
N_FEAT = 16
RES = 512

NC = 2
NS = 16
NW = NC * NS
LANES = 16

PER_W = N_PTS // NW
B = 128
NCH = PER_W // B


_BCAST_DNUMS = lax.GatherDimensionNumbers(
    offset_dims=(), collapsed_slice_dims=(0,), start_index_map=(0,))


def _lane_bcast(vec, lane_idx):
    return lax.gather(
        vec, lane_idx[:, None], _BCAST_DNUMS, (1,),
        mode=lax.GatherScatterMode.PROMISE_IN_BOUNDS)


def _tri_body(coords_hbm, table_hbm, out_hbm, cbuf, idx, wbuf, gbuf, obuf, sem):
    wid = lax.axis_index("s") * NC + lax.axis_index("c")
    base0 = wid * PER_W

    def chunk_body(ch, carry):
        base = base0 + ch * B
        pltpu.sync_copy(coords_hbm.at[:, pl.ds(base, B)], cbuf)

        def ixw_body(g, carry2):
            sl = pl.ds(g * LANES, LANES)
            for p in range(3):
                d0, d1 = ((0, 2), (0, 1), (1, 2))[p]
                cx = cbuf[d0, sl]
                cy = cbuf[d1, sl]
                fx = jnp.clip(cx * (RES - 1.0), 0.0, RES - 1.0)
                fy = jnp.clip(cy * (RES - 1.0), 0.0, RES - 1.0)
                ix0 = fx.astype(jnp.int32)
                iy0 = fy.astype(jnp.int32)
                wx = fx - ix0.astype(jnp.float32)
                wy = fy - iy0.astype(jnp.float32)
                ix1 = jnp.minimum(ix0 + 1, RES - 1)
                iy1 = jnp.minimum(iy0 + 1, RES - 1)
                pbase = p * RES * RES
                r0 = pbase + iy0 * RES
                r1 = pbase + iy1 * RES
                idx[4 * p + 0, sl] = r0 + ix0
                idx[4 * p + 1, sl] = r0 + ix1
                idx[4 * p + 2, sl] = r1 + ix0
                idx[4 * p + 3, sl] = r1 + ix1
                wbuf[2 * p + 0, sl] = wx
                wbuf[2 * p + 1, sl] = wy
            return carry2

        lax.fori_loop(0, B // LANES, ixw_body, 0)

        copies = [
            pltpu.async_copy(table_hbm.at[idx.at[j]], gbuf.at[j], sem)
            for j in range(12)
        ]
        for cp in copies:
            cp.wait()

        def blend_body(g, carry2):
            sl = pl.ds(g * LANES, LANES)
            wv = [wbuf[r, sl] for r in range(6)]
            for j in range(LANES):
                n = g * LANES + j
                jvec = jnp.full((LANES,), j, jnp.int32)
                for p in range(3):
                    wx = _lane_bcast(wv[2 * p + 0], jvec)
                    wy = _lane_bcast(wv[2 * p + 1], jvec)
                    f00 = gbuf[4 * p + 0, n]
                    f01 = gbuf[4 * p + 1, n]
                    f10 = gbuf[4 * p + 2, n]
                    f11 = gbuf[4 * p + 3, n]
                    top = f00 + wx * (f01 - f00)
                    bot = f10 + wx * (f11 - f10)
                    obuf[n, pl.ds(p * LANES, LANES)] = top + wy * (bot - top)
            return carry2

        lax.fori_loop(0, B // LANES, blend_body, 0)

        pltpu.sync_copy(obuf, out_hbm.at[pl.ds(base, B)])
        return carry

    lax.fori_loop(0, NCH, chunk_body, 0)


@jax.jit
def _tri_plane_sc(coords_t, table):
    mesh = plsc.VectorSubcoreMesh(
        core_axis_name="c", subcore_axis_name="s",
        num_cores=NC, num_subcores=NS)
    f = functools.partial(
        pl.kernel,
        out_type=jax.ShapeDtypeStruct((N_PTS, 3 * N_FEAT), jnp.float32),
        mesh=mesh,
        scratch_types=[
            pltpu.VMEM((3, B), jnp.float32),
            pltpu.VMEM((12, B), jnp.int32),
            pltpu.VMEM((6, B), jnp.float32),
            pltpu.VMEM((12, B, N_FEAT), jnp.float32),
            pltpu.VMEM((B, 3 * N_FEAT), jnp.float32),
            pltpu.SemaphoreType.DMA,
        ],
        compiler_params=pltpu.CompilerParams(use_tc_tiling_on_sc=False),
    )(_tri_body)
    return f(coords_t, table)


def kernel(coordinates, planes_features):
    coords_t = coordinates.T
    table = planes_features.transpose(0, 2, 3, 1).reshape(-1, N_FEAT)
    return _tri_plane_sc(coords_t, table)
